# baseline (speedup 1.0000x reference)
import functools

import numpy as np
import jax
import jax.numpy as jnp
from jax import lax
from jax.experimental import pallas as pl
from jax.experimental.pallas import tpu as pltpu
from jax.experimental.pallas import tpu_sc as plsc

_MAX_LEN = 5000
_NC = 2
_NS = 16
_NW = _NC * _NS


def _positional_encoding(max_len, d_model):
    pos = np.arange(max_len)[:, None].astype(np.float32)
    i = np.arange(d_model)[None, :].astype(np.float32)
    angle_rates = 1.0 / np.power(10000.0, (2.0 * np.floor(i / 2.0)) / np.float32(d_model))
    angle_rads = pos * angle_rates
    angle_rads[:, 0::2] = np.sin(angle_rads[:, 0::2])
    angle_rads[:, 1::2] = np.cos(angle_rads[:, 1::2])
    return angle_rads


@functools.partial(jax.jit, static_argnums=(3, 4, 5))
def _launch(idx_flat, E, pos, B, S, D):
    rows_per_w = B // _NW
    idx_per_w = rows_per_w * S
    scale = float(np.float32(np.sqrt(np.float32(D))))
    n_full = S // 128
    rem = S - n_full * 128

    mesh = plsc.VectorSubcoreMesh(
        core_axis_name="c", subcore_axis_name="s",
        num_cores=_NC, num_subcores=_NS)

    @functools.partial(
        pl.kernel,
        out_type=jax.ShapeDtypeStruct((B * S, D), jnp.float32),
        mesh=mesh,
        scratch_types=[
            pltpu.VMEM((idx_per_w,), jnp.int32),
            pltpu.VMEM((S, D), jnp.float32),
            pltpu.VMEM((S, D), jnp.float32),
            pltpu.SemaphoreType.DMA,
        ],
        compiler_params=pltpu.CompilerParams(use_tc_tiling_on_sc=False),
    )
    def run(idx_hbm, table_hbm, pos_hbm, out_hbm, idx_v, pos_v, rows_v, sem):
        wid = lax.axis_index("s") * _NC + lax.axis_index("c")
        base = wid * idx_per_w
        pltpu.sync_copy(idx_hbm.at[pl.ds(base, idx_per_w)], idx_v)
        pltpu.sync_copy(pos_hbm, pos_v)

        def row_body(r, carry):
            off = r * S
            copies = []
            for c in range(n_full):
                copies.append(pltpu.async_copy(
                    table_hbm.at[idx_v.at[pl.ds(off + c * 128, 128)]],
                    rows_v.at[pl.ds(c * 128, 128)], sem))
            if rem:
                copies.append(pltpu.async_copy(
                    table_hbm.at[idx_v.at[pl.ds(off + n_full * 128, rem)]],
                    rows_v.at[pl.ds(n_full * 128, rem)], sem))
            for cp in copies:
                cp.wait()

            def fma_body(j, carry2):
                for cch in range(D // 16):
                    sl = pl.ds(cch * 16, 16)
                    rows_v[j, sl] = rows_v[j, sl] * scale + pos_v[j, sl]
                return carry2
            lax.fori_loop(0, S, fma_body, 0)

            pltpu.sync_copy(rows_v, out_hbm.at[pl.ds(base + off, S)])
            return carry
        lax.fori_loop(0, rows_per_w, row_body, 0)

    return run(idx_flat, E, pos)


def kernel(input, E):
    B, S = input.shape
    V, D = E.shape
    pos = jnp.asarray(_positional_encoding(_MAX_LEN, D)[:S], dtype=jnp.float32)
    out_flat = _launch(input.reshape(B * S), E, pos, B, S, D)
    return out_flat.reshape(B, S, D)
